# baseline (speedup 1.0000x reference)
import jax
import jax.numpy as jnp
from jax import lax
from jax.experimental import pallas as pl
from jax.experimental.pallas import tpu as pltpu

N = 4096
D = 64
BR = 512
NB = N // BR
DEPTH = 3

_DN_NT = (((1,), (1,)), ((), ()))


def _body(x_ref, pinv0_ref, pinv1_ref, phi0_ref, phi1_ref,
          w0_ref, w1_ref, k0_ref, k1_ref,
          out_ref, buf0_ref, buf1_ref, xpt0_ref, xpt1_ref,
          yt0_ref, yt1_ref, sem):
    p = pl.program_id(0)
    i = pl.program_id(1)
    step = p * NB + i
    slot = lax.rem(step, DEPTH)

    def issue(gstep, dst_slot):
        phase = gstep // NB
        rows = pl.ds(lax.rem(gstep, NB) * BR, BR)

        @pl.when(phase == 0)
        def _():
            pltpu.make_async_copy(pinv0_ref.at[rows, :],
                                  buf0_ref.at[dst_slot], sem.at[0, dst_slot]).start()
            pltpu.make_async_copy(pinv1_ref.at[rows, :],
                                  buf1_ref.at[dst_slot], sem.at[1, dst_slot]).start()

        @pl.when(phase == 1)
        def _():
            pltpu.make_async_copy(phi0_ref.at[rows, :],
                                  buf0_ref.at[dst_slot], sem.at[0, dst_slot]).start()
            pltpu.make_async_copy(phi1_ref.at[rows, :],
                                  buf1_ref.at[dst_slot], sem.at[1, dst_slot]).start()

    @pl.when(step == 0)
    def _():
        issue(0, 0)
        issue(1, 1)
        xpt0_ref[...] = lax.dot_general(
            w0_ref[...], x_ref[...], (((0,), (1,)), ((), ())),
            preferred_element_type=jnp.float32)
        xpt1_ref[...] = lax.dot_general(
            w1_ref[...], x_ref[...], (((0,), (1,)), ((), ())),
            preferred_element_type=jnp.float32)

    @pl.when(step < 2 * NB - (DEPTH - 1))
    def _():
        issue(step + DEPTH - 1, lax.rem(step + DEPTH - 1, DEPTH))

    def wait(stream, sref):
        pltpu.make_async_copy(pinv0_ref.at[pl.ds(0, BR), :],
                              sref.at[slot], sem.at[stream, slot]).wait()

    cols = pl.ds(i * BR, BR)

    @pl.when(p == 0)
    def _():
        wait(0, buf0_ref)
        t0 = lax.dot_general(xpt0_ref[...], buf0_ref[slot], _DN_NT,
                             preferred_element_type=jnp.float32)
        yt0_ref[:, cols] = k0_ref[:, cols] * t0
        wait(1, buf1_ref)
        t1 = lax.dot_general(xpt1_ref[...], buf1_ref[slot], _DN_NT,
                             preferred_element_type=jnp.float32)
        yt1_ref[:, cols] = k1_ref[:, cols] * t1

    @pl.when(p == 1)
    def _():
        wait(0, buf0_ref)
        z0 = lax.dot_general(yt0_ref[...], buf0_ref[slot], _DN_NT,
                             preferred_element_type=jnp.float32)
        out_ref[0, :, :] = jnp.maximum(z0, 0.0).T
        wait(1, buf1_ref)
        z1 = lax.dot_general(yt1_ref[...], buf1_ref[slot], _DN_NT,
                             preferred_element_type=jnp.float32)
        out_ref[1, :, :] = jnp.maximum(z1, 0.0).T


def kernel(x, phi_inv_0, phi_0, phi_inv_1, phi_1, W0, W1, k0, k1):
    full = lambda shape: pl.BlockSpec(shape, lambda p, i: (0,) * len(shape))
    hbm = pl.BlockSpec(memory_space=pl.ANY)

    return pl.pallas_call(
        _body,
        grid=(2, NB),
        in_specs=[
            full((N, D)),
            hbm,
            hbm,
            hbm,
            hbm,
            full((D, D)),
            full((D, D)),
            full((1, N)),
            full((1, N)),
        ],
        out_specs=pl.BlockSpec((2, BR, D),
                               lambda p, i: (0, jnp.where(p == 1, i, 0), 0)),
        out_shape=jax.ShapeDtypeStruct((2, N, D), jnp.float32),
        scratch_shapes=[
            pltpu.VMEM((DEPTH, BR, N), jnp.float32),
            pltpu.VMEM((DEPTH, BR, N), jnp.float32),
            pltpu.VMEM((D, N), jnp.float32),
            pltpu.VMEM((D, N), jnp.float32),
            pltpu.VMEM((D, N), jnp.float32),
            pltpu.VMEM((D, N), jnp.float32),
            pltpu.SemaphoreType.DMA((2, DEPTH)),
        ],
    )(x, phi_inv_0, phi_inv_1, phi_0, phi_1, W0, W1,
      k0.reshape(1, N), k1.reshape(1, N))


# speedup vs baseline: 1.0318x; 1.0318x over previous
import jax
import jax.numpy as jnp
from jax import lax
from jax.experimental import pallas as pl
from jax.experimental.pallas import tpu as pltpu

N = 4096
D = 64
BR = 512
NB = N // BR

_DN_NT = (((1,), (1,)), ((), ()))


def _body(x_ref, pinv0_ref, pinv1_ref, phi0_ref, phi1_ref,
          w0_ref, w1_ref, k0_ref, k1_ref,
          out_ref, buf0_ref, buf1_ref, xpt0_ref, xpt1_ref,
          yt0_ref, yt1_ref, sem):
    p = pl.program_id(0)
    i = pl.program_id(1)
    step = p * NB + i
    slot = lax.rem(step, 2)
    nslot = lax.rem(step + 1, 2)

    def issue(phase, blk, dst_slot):
        rows = pl.ds(blk * BR, BR)

        @pl.when(phase == 0)
        def _():
            pltpu.make_async_copy(pinv0_ref.at[rows, :],
                                  buf0_ref.at[dst_slot], sem.at[0, dst_slot]).start()
            pltpu.make_async_copy(pinv1_ref.at[rows, :],
                                  buf1_ref.at[dst_slot], sem.at[1, dst_slot]).start()

        @pl.when(phase == 1)
        def _():
            pltpu.make_async_copy(phi0_ref.at[rows, :],
                                  buf0_ref.at[dst_slot], sem.at[0, dst_slot]).start()
            pltpu.make_async_copy(phi1_ref.at[rows, :],
                                  buf1_ref.at[dst_slot], sem.at[1, dst_slot]).start()

    @pl.when(step == 0)
    def _():
        issue(0, 0, 0)
        xpt0_ref[...] = lax.dot_general(
            w0_ref[...], x_ref[...], (((0,), (1,)), ((), ())),
            preferred_element_type=jnp.float32)
        xpt1_ref[...] = lax.dot_general(
            w1_ref[...], x_ref[...], (((0,), (1,)), ((), ())),
            preferred_element_type=jnp.float32)

    @pl.when(step < 2 * NB - 1)
    def _():
        nstep = step + 1
        issue(nstep // NB, lax.rem(nstep, NB), nslot)

    def wait(stream, sref):
        pltpu.make_async_copy(pinv0_ref.at[pl.ds(0, BR), :],
                              sref.at[slot], sem.at[stream, slot]).wait()

    cols = pl.ds(i * BR, BR)

    @pl.when(p == 0)
    def _():
        wait(0, buf0_ref)
        t0 = lax.dot_general(xpt0_ref[...], buf0_ref[slot], _DN_NT,
                             preferred_element_type=jnp.float32)
        yt0_ref[:, cols] = k0_ref[:, cols] * t0
        wait(1, buf1_ref)
        t1 = lax.dot_general(xpt1_ref[...], buf1_ref[slot], _DN_NT,
                             preferred_element_type=jnp.float32)
        yt1_ref[:, cols] = k1_ref[:, cols] * t1

    @pl.when(p == 1)
    def _():
        wait(0, buf0_ref)
        z0 = lax.dot_general(yt0_ref[...], buf0_ref[slot], _DN_NT,
                             preferred_element_type=jnp.float32)
        out_ref[0, :, :] = jnp.maximum(z0, 0.0).T
        wait(1, buf1_ref)
        z1 = lax.dot_general(yt1_ref[...], buf1_ref[slot], _DN_NT,
                             preferred_element_type=jnp.float32)
        out_ref[1, :, :] = jnp.maximum(z1, 0.0).T


def kernel(x, phi_inv_0, phi_0, phi_inv_1, phi_1, W0, W1, k0, k1):
    full = lambda shape: pl.BlockSpec(shape, lambda p, i: (0,) * len(shape))
    hbm = pl.BlockSpec(memory_space=pl.ANY)

    return pl.pallas_call(
        _body,
        grid=(2, NB),
        in_specs=[
            full((N, D)),
            hbm,
            hbm,
            hbm,
            hbm,
            full((D, D)),
            full((D, D)),
            full((1, N)),
            full((1, N)),
        ],
        out_specs=pl.BlockSpec((2, BR, D),
                               lambda p, i: (0, jnp.where(p == 1, i, 0), 0)),
        out_shape=jax.ShapeDtypeStruct((2, N, D), jnp.float32),
        scratch_shapes=[
            pltpu.VMEM((2, BR, N), jnp.float32),
            pltpu.VMEM((2, BR, N), jnp.float32),
            pltpu.VMEM((D, N), jnp.float32),
            pltpu.VMEM((D, N), jnp.float32),
            pltpu.VMEM((D, N), jnp.float32),
            pltpu.VMEM((D, N), jnp.float32),
            pltpu.SemaphoreType.DMA((2, 2)),
        ],
    )(x, phi_inv_0, phi_inv_1, phi_0, phi_1, W0, W1,
      k0.reshape(1, N), k1.reshape(1, N))
